# baseline (speedup 1.0000x reference)
import jax
import jax.numpy as jnp
from jax.experimental import pallas as pl
from jax.experimental.pallas import tpu as pltpu

_BATCH = 1024
_CONN = 128


def _row_copy(at_ref, idx_ref, bufs, sems, j):
    return pltpu.make_async_copy(
        at_ref.at[pl.ds(idx_ref[j], 1), :],
        bufs.at[j],
        sems.at[j],
    )


def _body(idx_ref, at_ref, w_ref, sens_ref, thr_ref, o_ref, bufs, sems):
    for j in range(_CONN):
        _row_copy(at_ref, idx_ref, bufs, sems, j).start()
    acc = jnp.zeros((1, _BATCH), jnp.float32)
    for j in range(_CONN):
        _row_copy(at_ref, idx_ref, bufs, sems, j).wait()
        acc = acc + bufs[j] * w_ref[j]
    z = acc[0]
    z = z * sens_ref[0] - thr_ref[0]
    o_ref[...] = 1.0 / (1.0 + jnp.exp(-z))


def kernel(x, all_activations, connection_weights, sensitivity, threshold,
           connection_indices):
    del x
    return pl.pallas_call(
        _body,
        grid_spec=pltpu.PrefetchScalarGridSpec(
            num_scalar_prefetch=1,
            in_specs=[
                pl.BlockSpec(memory_space=pl.ANY),
                pl.BlockSpec(memory_space=pltpu.SMEM),
                pl.BlockSpec(memory_space=pltpu.SMEM),
                pl.BlockSpec(memory_space=pltpu.SMEM),
            ],
            out_specs=pl.BlockSpec(memory_space=pltpu.VMEM),
            scratch_shapes=[
                pltpu.VMEM((_CONN, 1, _BATCH), jnp.float32),
                pltpu.SemaphoreType.DMA((_CONN,)),
            ],
        ),
        out_shape=jax.ShapeDtypeStruct((_BATCH,), jnp.float32),
    )(connection_indices, all_activations.T, connection_weights,
      sensitivity, threshold)
